# baseline (speedup 1.0000x reference)
import functools

import jax
import jax.numpy as jnp
from jax import lax
from jax.experimental import pallas as pl
from jax.experimental.pallas import tpu as pltpu
from jax.experimental.pallas import tpu_sc as plsc

D = 128
NG = 50000
NM = 10000
E = 320000
IN_GRID = 96
GEO = 32

_NC = 2
_NS = 16
_NW = _NC * _NS
_EPW = E // _NW
_C = 80
_G = 5
_GRP = _C * _G
_NIT = _EPW // _GRP
_NMP = 10240
_MPW = _NMP // _NS


def _ln(y, g, b):
    mu = jnp.mean(y, axis=-1, keepdims=True)
    var = jnp.mean((y - mu) * (y - mu), axis=-1, keepdims=True)
    return (y - mu) * lax.rsqrt(var + 1e-5) * g + b


def _silu(x):
    return x * jax.nn.sigmoid(x)


def _dot(a, b):
    return jnp.dot(a, b, preferred_element_type=jnp.float32)


def _dott(at, b):
    return lax.dot_general(at, b, (((0,), (0,)), ((), ())),
                           preferred_element_type=jnp.float32)


def _grid_embed_body(xt_ref, gt_ref, w1x_ref, w1g_ref, b1_ref, w2_ref, b2_ref,
                     lg_ref, lb_ref, gw1_ref, gb1_ref, gw2_ref, gb2_ref,
                     glg_ref, glb_ref, o_ref, o2_ref):
    h = _silu(_dott(xt_ref[...], w1x_ref[...])
              + _dott(gt_ref[...], w1g_ref[...]) + b1_ref[...])
    y = _dot(h, w2_ref[...]) + b2_ref[...]
    glat = _ln(y, lg_ref[...], lb_ref[...])
    o_ref[...] = glat
    h2 = _silu(_dot(glat, gw1_ref[...]) + gb1_ref[...])
    y2 = _dot(h2, gw2_ref[...]) + gb2_ref[...]
    o2_ref[...] = _ln(y2, glg_ref[...], glb_ref[...]) + glat


def _small_embed_body(xt_ref, w1_ref, b1_ref, w2_ref, b2_ref, lg_ref, lb_ref,
                      o_ref):
    h = _silu(_dott(xt_ref[...], w1_ref[...]) + b1_ref[...])
    y = _dot(h, w2_ref[...]) + b2_ref[...]
    o_ref[...] = _ln(y, lg_ref[...], lb_ref[...])


def _edge_update_body(eft_ref, sf_ref, df_ref,
                      ew1_ref, eb1_ref, ew2_ref, eb2_ref, elg_ref, elb_ref,
                      mw1e_ref, mw1s_ref, mw1d_ref, mb1_ref, mw2_ref, mb2_ref,
                      mlg_ref, mlb_ref, o_ref):
    h0 = _silu(_dott(eft_ref[...], ew1_ref[...]) + eb1_ref[...])
    e0 = _ln(_dot(h0, ew2_ref[...]) + eb2_ref[...], elg_ref[...], elb_ref[...])
    bf = jnp.bfloat16
    h = _silu(_dot(e0.astype(bf), mw1e_ref[...].astype(bf))
              + _dot(sf_ref[...].astype(bf), mw1s_ref[...].astype(bf))
              + _dot(df_ref[...].astype(bf), mw1d_ref[...].astype(bf))
              + mb1_ref[...])
    y = _dot(h.astype(bf), mw2_ref[...].astype(bf)) + mb2_ref[...]
    o_ref[...] = e0 + _ln(y, mlg_ref[...], mlb_ref[...])


def _mesh_update_body(m_ref, a0_ref, a1_ref, w1m_ref, w1a_ref, b1_ref,
                      w2_ref, b2_ref, lg_ref, lb_ref, o_ref):
    a = a0_ref[0] + a1_ref[0]
    h = _silu(_dot(m_ref[...], w1m_ref[...]) + _dot(a, w1a_ref[...])
              + b1_ref[...])
    y = _dot(h, w2_ref[...]) + b2_ref[...]
    o_ref[...] = m_ref[...] + _ln(y, lg_ref[...], lb_ref[...])


def _grid_out_body(g_ref, w1_ref, b1_ref, w2_ref, b2_ref, lg_ref, lb_ref,
                   o_ref):
    h = _silu(_dot(g_ref[...], w1_ref[...]) + b1_ref[...])
    y = _dot(h, w2_ref[...]) + b2_ref[...]
    o_ref[...] = _ln(y, lg_ref[...], lb_ref[...]) + g_ref[...]


def _wspec(*dims):
    return pl.BlockSpec(dims, lambda i: (0,) * len(dims))


def _rspec(r, c):
    return pl.BlockSpec((r, c), lambda i: (i, 0))


def _cspec(c, r):
    return pl.BlockSpec((c, r), lambda i: (0, i))


def _run_rows(body, col_ins, row_ins, w_ins, n_rows, r):
    in_specs = ([_cspec(a.shape[0], r) for a in col_ins]
                + [_rspec(r, a.shape[1]) for a in row_ins]
                + [_wspec(*w.shape) for w in w_ins])
    return pl.pallas_call(
        body,
        grid=(pl.cdiv(n_rows, r),),
        in_specs=in_specs,
        out_specs=_rspec(r, D),
        out_shape=jax.ShapeDtypeStruct((n_rows, D), jnp.float32),
    )(*col_ins, *row_ins, *w_ins)


def _sc_mesh():
    return plsc.VectorSubcoreMesh(core_axis_name="c", subcore_axis_name="s",
                                  num_cores=_NC, num_subcores=_NS)


_EPT = E // _NS
_NIT2 = _EPT // _GRP


def _sc_gather(glat, mlat, src, dst):
    mesh = _sc_mesh()

    @functools.partial(
        pl.kernel,
        out_type=(jax.ShapeDtypeStruct((E, D), jnp.float32),
                  jax.ShapeDtypeStruct((E, D), jnp.float32)),
        mesh=mesh,
        scratch_types=[
            pltpu.VMEM((_EPT,), jnp.int32),
            pltpu.VMEM((_GRP, D), jnp.float32),
            pltpu.VMEM((_GRP, D), jnp.float32),
            pltpu.SemaphoreType.DMA,
            pltpu.SemaphoreType.DMA,
            pltpu.SemaphoreType.DMA,
        ],
    )
    def k(glat_hbm, mlat_hbm, src_hbm, dst_hbm, srcf_hbm, dstf_hbm,
          idx_v, r0, r1, sem_g, sem_w0, sem_w1):
        cid = lax.axis_index("c")
        sid = lax.axis_index("s")
        base0 = sid * _EPT

        def fire(tab, g, buf):
            for j in range(_G):
                pltpu.async_copy(
                    tab.at[idx_v.at[pl.ds(g * _GRP + j * _C, _C)]],
                    buf.at[pl.ds(j * _C, _C)], sem_g)

        def wait_g(tab, buf):
            for j in range(_G):
                pltpu.make_async_copy(
                    tab.at[idx_v.at[pl.ds(j * _C, _C)]],
                    buf.at[pl.ds(j * _C, _C)], sem_g).wait()

        def run(tab, idx_hbm, out_hbm):
            pltpu.sync_copy(idx_hbm.at[pl.ds(base0, _EPT)], idx_v)
            fire(tab, 0, r0)

            def pair(p, carry):
                g0 = 2 * p
                wait_g(tab, r0)
                pltpu.async_copy(r0, out_hbm.at[pl.ds(base0 + g0 * _GRP,
                                                      _GRP)], sem_w0)

                @pl.when(p > 0)
                def _():
                    pltpu.make_async_copy(
                        r1, out_hbm.at[pl.ds(base0, _GRP)], sem_w1).wait()

                fire(tab, g0 + 1, r1)
                wait_g(tab, r1)
                pltpu.make_async_copy(
                    r0, out_hbm.at[pl.ds(base0, _GRP)], sem_w0).wait()

                @pl.when(p < _NIT2 // 2 - 1)
                def _():
                    fire(tab, g0 + 2, r0)

                pltpu.async_copy(r1, out_hbm.at[pl.ds(base0 + (g0 + 1) * _GRP,
                                                      _GRP)], sem_w1)
                return carry

            lax.fori_loop(0, _NIT2 // 2, pair, 0)
            pltpu.make_async_copy(
                r1, out_hbm.at[pl.ds(base0, _GRP)], sem_w1).wait()

        @pl.when(cid == 0)
        def _():
            run(glat_hbm, src_hbm, srcf_hbm)

        @pl.when(cid == 1)
        def _():
            run(mlat_hbm, dst_hbm, dstf_hbm)

    return k(glat, mlat, src, dst)


def _sc_segsum(eupd, dst, zeros_slice):
    mesh = _sc_mesh()

    @functools.partial(
        pl.kernel,
        out_type=jax.ShapeDtypeStruct((_NC, _NMP, D), jnp.float32),
        mesh=mesh,
        scratch_types=[
            pltpu.VMEM_SHARED((_NMP, D), jnp.float32),
            pltpu.VMEM((_EPW,), jnp.int32),
            pltpu.VMEM((_C, D), jnp.float32),
            pltpu.VMEM((_C, D), jnp.float32),
            pltpu.SemaphoreType.DMA,
        ],
    )
    def k(eupd_hbm, dst_hbm, z_hbm, agg_hbm, shared, di_v, q0, q1, sem_l):
        cid = lax.axis_index("c")
        sid = lax.axis_index("s")
        wid = sid * _NC + cid
        pltpu.sync_copy(z_hbm, shared.at[pl.ds(sid * _MPW, _MPW)])
        plsc.subcore_barrier()
        base0 = wid * _EPW
        pltpu.sync_copy(dst_hbm.at[pl.ds(base0, _EPW)], di_v)
        ngrp = _EPW // _C

        def load(g, buf):
            pltpu.async_copy(eupd_hbm.at[pl.ds(base0 + g * _C, _C)], buf,
                             sem_l)

        def wait_l(buf):
            pltpu.make_async_copy(eupd_hbm.at[pl.ds(base0, _C)], buf,
                                  sem_l).wait()

        def scat(g, buf):
            pltpu.sync_copy(buf, shared.at[di_v.at[pl.ds(g * _C, _C)]],
                            add=True)

        load(0, q0)

        def pairbody(p, carry):
            g0 = 2 * p
            load(g0 + 1, q1)
            wait_l(q0)
            scat(g0, q0)
            load(g0 + 2, q0)
            wait_l(q1)
            scat(g0 + 1, q1)
            return carry

        lax.fori_loop(0, ngrp // 2, pairbody, 0)
        wait_l(q0)
        scat(ngrp - 1, q0)
        plsc.subcore_barrier()
        pltpu.sync_copy(shared.at[pl.ds(sid * _MPW, _MPW)],
                        agg_hbm.at[cid, pl.ds(sid * _MPW, _MPW)])

    return k(eupd, dst, zeros_slice)


def kernel(grid_nodes_features, params, edge_index):
    b = grid_nodes_features.shape[0]
    gx = grid_nodes_features.reshape(NG, IN_GRID)

    ge = params["grid_embed"]
    me = params["mesh_embed"]
    ee = params["edge_embed"]
    em = params["edge_mlp"]
    nm = params["node_mlp"]
    gn = params["grid_node_mlp"]

    def row(v):
        return v.reshape(1, D)

    gw1 = ge["W1"]
    gew = [gw1[:IN_GRID], gw1[IN_GRID:], row(ge["b1"]), ge["W2"],
           row(ge["b2"]), row(ge["ln_g"]), row(ge["ln_b"]),
           gn["W1"], row(gn["b1"]), gn["W2"], row(gn["b2"]), row(gn["ln_g"]),
           row(gn["ln_b"])]
    grid_lat, grid_out = pl.pallas_call(
        _grid_embed_body,
        grid=(pl.cdiv(NG, 4096),),
        in_specs=([_cspec(IN_GRID, 4096), _cspec(GEO, 4096)]
                  + [_wspec(*w.shape) for w in gew]),
        out_specs=(_rspec(4096, D), _rspec(4096, D)),
        out_shape=(jax.ShapeDtypeStruct((NG, D), jnp.float32),
                   jax.ShapeDtypeStruct((NG, D), jnp.float32)),
    )(gx.T, params["grid_geo"].T, *gew)

    mesh_lat = _run_rows(
        _small_embed_body,
        [params["mesh_geo"].T], [],
        [me["W1"], row(me["b1"]), me["W2"], row(me["b2"]), row(me["ln_g"]),
         row(me["ln_b"])],
        NM, 2048)

    src_f, dst_f = _sc_gather(grid_lat, mesh_lat, edge_index[0], edge_index[1])

    emw1 = em["W1"]
    e_upd = _run_rows(
        _edge_update_body,
        [params["edge_feats"].T], [src_f, dst_f],
        [ee["W1"], row(ee["b1"]), ee["W2"], row(ee["b2"]), row(ee["ln_g"]),
         row(ee["ln_b"]),
         emw1[:D], emw1[D:2 * D], emw1[2 * D:], row(em["b1"]), em["W2"],
         row(em["b2"]), row(em["ln_g"]), row(em["ln_b"])],
        E, 4096)

    zeros_slice = jnp.zeros((_MPW, D), jnp.float32)
    agg = _sc_segsum(e_upd, edge_index[1], zeros_slice)

    nw1 = nm["W1"]
    nmw = [nw1[:D], nw1[D:], row(nm["b1"]), nm["W2"], row(nm["b2"]),
           row(nm["ln_g"]), row(nm["ln_b"])]
    mesh_out = pl.pallas_call(
        _mesh_update_body,
        grid=(NM // 1000,),
        in_specs=([_rspec(1000, D),
                   pl.BlockSpec((1, 1000, D), lambda i: (0, i, 0)),
                   pl.BlockSpec((1, 1000, D), lambda i: (1, i, 0))]
                  + [_wspec(*w.shape) for w in nmw]),
        out_specs=_rspec(1000, D),
        out_shape=jax.ShapeDtypeStruct((NM, D), jnp.float32),
    )(mesh_lat, agg, agg, *nmw)

    return (grid_out.reshape(b, NG, D), mesh_out.reshape(b, NM, D))


# speedup vs baseline: 1.0117x; 1.0117x over previous
import functools

import jax
import jax.numpy as jnp
from jax import lax
from jax.experimental import pallas as pl
from jax.experimental.pallas import tpu as pltpu
from jax.experimental.pallas import tpu_sc as plsc

D = 128
NG = 50000
NM = 10000
E = 320000
IN_GRID = 96
GEO = 32

_NC = 2
_NS = 16
_NW = _NC * _NS
_EPW = E // _NW
_C = 80
_G = 5
_GRP = _C * _G
_NIT = _EPW // _GRP
_NMP = 10240
_MPW = _NMP // _NS


def _ln(y, g, b):
    mu = jnp.mean(y, axis=-1, keepdims=True)
    var = jnp.mean((y - mu) * (y - mu), axis=-1, keepdims=True)
    return (y - mu) * lax.rsqrt(var + 1e-5) * g + b


def _silu(x):
    return x * jax.nn.sigmoid(x)


def _dot(a, b):
    return jnp.dot(a, b, preferred_element_type=jnp.float32)


def _dott(at, b):
    return lax.dot_general(at, b, (((0,), (0,)), ((), ())),
                           preferred_element_type=jnp.float32)


def _grid_embed_body(xt_ref, gt_ref, w1x_ref, w1g_ref, b1_ref, w2_ref, b2_ref,
                     lg_ref, lb_ref, o_ref):
    h = _silu(_dott(xt_ref[...], w1x_ref[...])
              + _dott(gt_ref[...], w1g_ref[...]) + b1_ref[...])
    y = _dot(h, w2_ref[...]) + b2_ref[...]
    o_ref[...] = _ln(y, lg_ref[...], lb_ref[...])


def _small_embed_body(xt_ref, w1_ref, b1_ref, w2_ref, b2_ref, lg_ref, lb_ref,
                      o_ref):
    h = _silu(_dott(xt_ref[...], w1_ref[...]) + b1_ref[...])
    y = _dot(h, w2_ref[...]) + b2_ref[...]
    o_ref[...] = _ln(y, lg_ref[...], lb_ref[...])


def _edge_update_body(eft_ref, sf_ref, df_ref,
                      ew1_ref, eb1_ref, ew2_ref, eb2_ref, elg_ref, elb_ref,
                      mw1e_ref, mw1s_ref, mw1d_ref, mb1_ref, mw2_ref, mb2_ref,
                      mlg_ref, mlb_ref, o_ref):
    h0 = _silu(_dott(eft_ref[...], ew1_ref[...]) + eb1_ref[...])
    e0 = _ln(_dot(h0, ew2_ref[...]) + eb2_ref[...], elg_ref[...], elb_ref[...])
    bf = jnp.bfloat16
    h = _silu(_dot(e0.astype(bf), mw1e_ref[...].astype(bf))
              + _dot(sf_ref[...].astype(bf), mw1s_ref[...].astype(bf))
              + _dot(df_ref[...].astype(bf), mw1d_ref[...].astype(bf))
              + mb1_ref[...])
    y = _dot(h.astype(bf), mw2_ref[...].astype(bf)) + mb2_ref[...]
    o_ref[...] = e0 + _ln(y, mlg_ref[...], mlb_ref[...])


def _mesh_update_body(m_ref, a0_ref, a1_ref, w1m_ref, w1a_ref, b1_ref,
                      w2_ref, b2_ref, lg_ref, lb_ref, o_ref):
    a = a0_ref[0] + a1_ref[0]
    h = _silu(_dot(m_ref[...], w1m_ref[...]) + _dot(a, w1a_ref[...])
              + b1_ref[...])
    y = _dot(h, w2_ref[...]) + b2_ref[...]
    o_ref[...] = m_ref[...] + _ln(y, lg_ref[...], lb_ref[...])


def _grid_out_body(g_ref, w1_ref, b1_ref, w2_ref, b2_ref, lg_ref, lb_ref,
                   o_ref):
    h = _silu(_dot(g_ref[...], w1_ref[...]) + b1_ref[...])
    y = _dot(h, w2_ref[...]) + b2_ref[...]
    o_ref[...] = _ln(y, lg_ref[...], lb_ref[...]) + g_ref[...]


def _wspec(*dims):
    return pl.BlockSpec(dims, lambda i: (0,) * len(dims))


def _rspec(r, c):
    return pl.BlockSpec((r, c), lambda i: (i, 0))


def _cspec(c, r):
    return pl.BlockSpec((c, r), lambda i: (0, i))


def _run_rows(body, col_ins, row_ins, w_ins, n_rows, r):
    in_specs = ([_cspec(a.shape[0], r) for a in col_ins]
                + [_rspec(r, a.shape[1]) for a in row_ins]
                + [_wspec(*w.shape) for w in w_ins])
    return pl.pallas_call(
        body,
        grid=(pl.cdiv(n_rows, r),),
        in_specs=in_specs,
        out_specs=_rspec(r, D),
        out_shape=jax.ShapeDtypeStruct((n_rows, D), jnp.float32),
    )(*col_ins, *row_ins, *w_ins)


def _sc_mesh():
    return plsc.VectorSubcoreMesh(core_axis_name="c", subcore_axis_name="s",
                                  num_cores=_NC, num_subcores=_NS)


_EPT = E // _NS
_NIT2 = _EPT // _GRP


def _sc_gather(glat, mlat, src, dst):
    mesh = _sc_mesh()

    @functools.partial(
        pl.kernel,
        out_type=(jax.ShapeDtypeStruct((E, D), jnp.float32),
                  jax.ShapeDtypeStruct((E, D), jnp.float32)),
        mesh=mesh,
        scratch_types=[
            pltpu.VMEM((_EPT,), jnp.int32),
            pltpu.VMEM((_GRP, D), jnp.float32),
            pltpu.VMEM((_GRP, D), jnp.float32),
            pltpu.SemaphoreType.DMA,
            pltpu.SemaphoreType.DMA,
            pltpu.SemaphoreType.DMA,
        ],
    )
    def k(glat_hbm, mlat_hbm, src_hbm, dst_hbm, srcf_hbm, dstf_hbm,
          idx_v, r0, r1, sem_g, sem_w0, sem_w1):
        cid = lax.axis_index("c")
        sid = lax.axis_index("s")
        base0 = sid * _EPT

        def fire(tab, g, buf):
            for j in range(_G):
                pltpu.async_copy(
                    tab.at[idx_v.at[pl.ds(g * _GRP + j * _C, _C)]],
                    buf.at[pl.ds(j * _C, _C)], sem_g)

        def wait_g(tab, buf):
            for j in range(_G):
                pltpu.make_async_copy(
                    tab.at[idx_v.at[pl.ds(j * _C, _C)]],
                    buf.at[pl.ds(j * _C, _C)], sem_g).wait()

        def run(tab, idx_hbm, out_hbm):
            pltpu.sync_copy(idx_hbm.at[pl.ds(base0, _EPT)], idx_v)
            fire(tab, 0, r0)

            def pair(p, carry):
                g0 = 2 * p
                wait_g(tab, r0)
                pltpu.async_copy(r0, out_hbm.at[pl.ds(base0 + g0 * _GRP,
                                                      _GRP)], sem_w0)

                @pl.when(p > 0)
                def _():
                    pltpu.make_async_copy(
                        r1, out_hbm.at[pl.ds(base0, _GRP)], sem_w1).wait()

                fire(tab, g0 + 1, r1)
                wait_g(tab, r1)
                pltpu.make_async_copy(
                    r0, out_hbm.at[pl.ds(base0, _GRP)], sem_w0).wait()

                @pl.when(p < _NIT2 // 2 - 1)
                def _():
                    fire(tab, g0 + 2, r0)

                pltpu.async_copy(r1, out_hbm.at[pl.ds(base0 + (g0 + 1) * _GRP,
                                                      _GRP)], sem_w1)
                return carry

            lax.fori_loop(0, _NIT2 // 2, pair, 0)
            pltpu.make_async_copy(
                r1, out_hbm.at[pl.ds(base0, _GRP)], sem_w1).wait()

        @pl.when(cid == 0)
        def _():
            run(glat_hbm, src_hbm, srcf_hbm)

        @pl.when(cid == 1)
        def _():
            run(mlat_hbm, dst_hbm, dstf_hbm)

    return k(glat, mlat, src, dst)


def _sc_segsum(eupd, dst, zeros_slice):
    mesh = _sc_mesh()

    @functools.partial(
        pl.kernel,
        out_type=jax.ShapeDtypeStruct((_NC, _NMP, D), jnp.float32),
        mesh=mesh,
        scratch_types=[
            pltpu.VMEM_SHARED((_NMP, D), jnp.float32),
            pltpu.VMEM((_EPW,), jnp.int32),
            pltpu.VMEM((_C, D), jnp.float32),
            pltpu.VMEM((_C, D), jnp.float32),
            pltpu.SemaphoreType.DMA,
        ],
    )
    def k(eupd_hbm, dst_hbm, z_hbm, agg_hbm, shared, di_v, q0, q1, sem_l):
        cid = lax.axis_index("c")
        sid = lax.axis_index("s")
        wid = sid * _NC + cid
        pltpu.sync_copy(z_hbm, shared.at[pl.ds(sid * _MPW, _MPW)])
        plsc.subcore_barrier()
        base0 = wid * _EPW
        pltpu.sync_copy(dst_hbm.at[pl.ds(base0, _EPW)], di_v)
        ngrp = _EPW // _C

        def load(g, buf):
            pltpu.async_copy(eupd_hbm.at[pl.ds(base0 + g * _C, _C)], buf,
                             sem_l)

        def wait_l(buf):
            pltpu.make_async_copy(eupd_hbm.at[pl.ds(base0, _C)], buf,
                                  sem_l).wait()

        def scat(g, buf):
            pltpu.sync_copy(buf, shared.at[di_v.at[pl.ds(g * _C, _C)]],
                            add=True)

        load(0, q0)

        def pairbody(p, carry):
            g0 = 2 * p
            load(g0 + 1, q1)
            wait_l(q0)
            scat(g0, q0)
            load(g0 + 2, q0)
            wait_l(q1)
            scat(g0 + 1, q1)
            return carry

        lax.fori_loop(0, ngrp // 2, pairbody, 0)
        wait_l(q0)
        scat(ngrp - 1, q0)
        plsc.subcore_barrier()
        pltpu.sync_copy(shared.at[pl.ds(sid * _MPW, _MPW)],
                        agg_hbm.at[cid, pl.ds(sid * _MPW, _MPW)])

    return k(eupd, dst, zeros_slice)


def kernel(grid_nodes_features, params, edge_index):
    b = grid_nodes_features.shape[0]
    gx = grid_nodes_features.reshape(NG, IN_GRID)

    ge = params["grid_embed"]
    me = params["mesh_embed"]
    ee = params["edge_embed"]
    em = params["edge_mlp"]
    nm = params["node_mlp"]
    gn = params["grid_node_mlp"]

    def row(v):
        return v.reshape(1, D)

    gw1 = ge["W1"]
    grid_lat = _run_rows(
        _grid_embed_body,
        [gx.T, params["grid_geo"].T], [],
        [gw1[:IN_GRID], gw1[IN_GRID:], row(ge["b1"]), ge["W2"], row(ge["b2"]),
         row(ge["ln_g"]), row(ge["ln_b"])],
        NG, 4096)

    mesh_lat = _run_rows(
        _small_embed_body,
        [params["mesh_geo"].T], [],
        [me["W1"], row(me["b1"]), me["W2"], row(me["b2"]), row(me["ln_g"]),
         row(me["ln_b"])],
        NM, 2048)

    src_f, dst_f = _sc_gather(grid_lat, mesh_lat, edge_index[0], edge_index[1])

    emw1 = em["W1"]
    e_upd = _run_rows(
        _edge_update_body,
        [params["edge_feats"].T], [src_f, dst_f],
        [ee["W1"], row(ee["b1"]), ee["W2"], row(ee["b2"]), row(ee["ln_g"]),
         row(ee["ln_b"]),
         emw1[:D], emw1[D:2 * D], emw1[2 * D:], row(em["b1"]), em["W2"],
         row(em["b2"]), row(em["ln_g"]), row(em["ln_b"])],
        E, 4096)

    zeros_slice = jnp.zeros((_MPW, D), jnp.float32)
    agg = _sc_segsum(e_upd, edge_index[1], zeros_slice)

    nw1 = nm["W1"]
    nmw = [nw1[:D], nw1[D:], row(nm["b1"]), nm["W2"], row(nm["b2"]),
           row(nm["ln_g"]), row(nm["ln_b"])]
    mesh_out = pl.pallas_call(
        _mesh_update_body,
        grid=(NM // 1000,),
        in_specs=([_rspec(1000, D),
                   pl.BlockSpec((1, 1000, D), lambda i: (0, i, 0)),
                   pl.BlockSpec((1, 1000, D), lambda i: (1, i, 0))]
                  + [_wspec(*w.shape) for w in nmw]),
        out_specs=_rspec(1000, D),
        out_shape=jax.ShapeDtypeStruct((NM, D), jnp.float32),
    )(mesh_lat, agg, agg, *nmw)

    grid_out = _run_rows(
        _grid_out_body,
        [], [grid_lat],
        [gn["W1"], row(gn["b1"]), gn["W2"], row(gn["b2"]), row(gn["ln_g"]),
         row(gn["ln_b"])],
        NG, 4096)

    return (grid_out.reshape(b, NG, D), mesh_out.reshape(b, NM, D))


# speedup vs baseline: 1.1566x; 1.1432x over previous
import functools

import jax
import jax.numpy as jnp
from jax import lax
from jax.experimental import pallas as pl
from jax.experimental.pallas import tpu as pltpu
from jax.experimental.pallas import tpu_sc as plsc

D = 128
NG = 50000
NM = 10000
E = 320000
IN_GRID = 96
GEO = 32

_NC = 2
_NS = 16
_NW = _NC * _NS
_EPW = E // _NW
_C = 80
_G = 5
_GRP = _C * _G
_NIT = _EPW // _GRP
_NMP = 10240
_MPW = _NMP // _NS
_CS = 40


def _ln(y, g, b):
    mu = jnp.mean(y, axis=-1, keepdims=True)
    var = jnp.mean((y - mu) * (y - mu), axis=-1, keepdims=True)
    return (y - mu) * lax.rsqrt(var + 1e-5) * g + b


def _silu(x):
    return x * jax.nn.sigmoid(x)


def _dot(a, b):
    return jnp.dot(a, b, preferred_element_type=jnp.float32)


def _dott(at, b):
    return lax.dot_general(at, b, (((0,), (0,)), ((), ())),
                           preferred_element_type=jnp.float32)


def _grid_embed_body(xt_ref, gt_ref, w1x_ref, w1g_ref, b1_ref, w2_ref, b2_ref,
                     lg_ref, lb_ref, o_ref):
    h = _silu(_dott(xt_ref[...], w1x_ref[...])
              + _dott(gt_ref[...], w1g_ref[...]) + b1_ref[...])
    y = _dot(h, w2_ref[...]) + b2_ref[...]
    o_ref[...] = _ln(y, lg_ref[...], lb_ref[...])


def _small_embed_body(xt_ref, w1_ref, b1_ref, w2_ref, b2_ref, lg_ref, lb_ref,
                      o_ref):
    h = _silu(_dott(xt_ref[...], w1_ref[...]) + b1_ref[...])
    y = _dot(h, w2_ref[...]) + b2_ref[...]
    o_ref[...] = _ln(y, lg_ref[...], lb_ref[...])


def _edge_update_body(eft_ref, sf_ref, df_ref,
                      ew1_ref, eb1_ref, ew2_ref, eb2_ref, elg_ref, elb_ref,
                      mw1e_ref, mw1s_ref, mw1d_ref, mb1_ref, mw2_ref, mb2_ref,
                      mlg_ref, mlb_ref, o_ref):
    h0 = _silu(_dott(eft_ref[...], ew1_ref[...]) + eb1_ref[...])
    e0 = _ln(_dot(h0, ew2_ref[...]) + eb2_ref[...], elg_ref[...], elb_ref[...])
    h = _silu(_dot(e0, mw1e_ref[...]) + _dot(sf_ref[...], mw1s_ref[...])
              + _dot(df_ref[...], mw1d_ref[...]) + mb1_ref[...])
    y = _dot(h, mw2_ref[...]) + mb2_ref[...]
    o_ref[...] = e0 + _ln(y, mlg_ref[...], mlb_ref[...])


def _mesh_update_body(m_ref, a0_ref, a1_ref, a2_ref, a3_ref, w1m_ref,
                      w1a_ref, b1_ref, w2_ref, b2_ref, lg_ref, lb_ref, o_ref):
    a = (a0_ref[0] + a1_ref[0]) + (a2_ref[0] + a3_ref[0])
    h = _silu(_dot(m_ref[...], w1m_ref[...]) + _dot(a, w1a_ref[...])
              + b1_ref[...])
    y = _dot(h, w2_ref[...]) + b2_ref[...]
    o_ref[...] = m_ref[...] + _ln(y, lg_ref[...], lb_ref[...])


def _grid_out_body(g_ref, w1_ref, b1_ref, w2_ref, b2_ref, lg_ref, lb_ref,
                   o_ref):
    h = _silu(_dot(g_ref[...], w1_ref[...]) + b1_ref[...])
    y = _dot(h, w2_ref[...]) + b2_ref[...]
    o_ref[...] = _ln(y, lg_ref[...], lb_ref[...]) + g_ref[...]


def _wspec(*dims):
    return pl.BlockSpec(dims, lambda i: (0,) * len(dims))


def _rspec(r, c):
    return pl.BlockSpec((r, c), lambda i: (i, 0))


def _cspec(c, r):
    return pl.BlockSpec((c, r), lambda i: (0, i))


def _run_rows(body, col_ins, row_ins, w_ins, n_rows, r):
    in_specs = ([_cspec(a.shape[0], r) for a in col_ins]
                + [_rspec(r, a.shape[1]) for a in row_ins]
                + [_wspec(*w.shape) for w in w_ins])
    return pl.pallas_call(
        body,
        grid=(pl.cdiv(n_rows, r),),
        in_specs=in_specs,
        out_specs=_rspec(r, D),
        out_shape=jax.ShapeDtypeStruct((n_rows, D), jnp.float32),
    )(*col_ins, *row_ins, *w_ins)


def _sc_mesh():
    return plsc.VectorSubcoreMesh(core_axis_name="c", subcore_axis_name="s",
                                  num_cores=_NC, num_subcores=_NS)


_ECH = E // 2
_EPT = _ECH // _NS
_NIT2 = _EPT // _GRP


def _sc_gather(glat, mlat, src, dst):
    mesh = _sc_mesh()

    @functools.partial(
        pl.kernel,
        out_type=(jax.ShapeDtypeStruct((_ECH, D), jnp.float32),
                  jax.ShapeDtypeStruct((_ECH, D), jnp.float32)),
        mesh=mesh,
        scratch_types=[
            pltpu.VMEM((_EPT,), jnp.int32),
            pltpu.VMEM((_GRP, D), jnp.float32),
            pltpu.VMEM((_GRP, D), jnp.float32),
            pltpu.SemaphoreType.DMA,
            pltpu.SemaphoreType.DMA,
            pltpu.SemaphoreType.DMA,
        ],
    )
    def k(glat_hbm, mlat_hbm, src_hbm, dst_hbm, srcf_hbm, dstf_hbm,
          idx_v, r0, r1, sem_g, sem_w0, sem_w1):
        cid = lax.axis_index("c")
        sid = lax.axis_index("s")
        base0 = sid * _EPT

        def fire(tab, g, buf):
            for j in range(_G):
                pltpu.async_copy(
                    tab.at[idx_v.at[pl.ds(g * _GRP + j * _C, _C)]],
                    buf.at[pl.ds(j * _C, _C)], sem_g)

        def wait_g(tab, buf):
            for j in range(_G):
                pltpu.make_async_copy(
                    tab.at[idx_v.at[pl.ds(j * _C, _C)]],
                    buf.at[pl.ds(j * _C, _C)], sem_g).wait()

        def run(tab, idx_hbm, out_hbm):
            pltpu.sync_copy(idx_hbm.at[pl.ds(base0, _EPT)], idx_v)
            fire(tab, 0, r0)

            def pair(p, carry):
                g0 = 2 * p
                wait_g(tab, r0)
                pltpu.async_copy(r0, out_hbm.at[pl.ds(base0 + g0 * _GRP,
                                                      _GRP)], sem_w0)

                @pl.when(p > 0)
                def _():
                    pltpu.make_async_copy(
                        r1, out_hbm.at[pl.ds(base0, _GRP)], sem_w1).wait()

                fire(tab, g0 + 1, r1)
                wait_g(tab, r1)
                pltpu.make_async_copy(
                    r0, out_hbm.at[pl.ds(base0, _GRP)], sem_w0).wait()

                fire(tab, g0 + 2, r0)
                pltpu.async_copy(r1, out_hbm.at[pl.ds(base0 + (g0 + 1) * _GRP,
                                                      _GRP)], sem_w1)
                return carry

            lax.fori_loop(0, _NIT2 // 2, pair, 0)
            wait_g(tab, r0)
            pltpu.make_async_copy(
                r1, out_hbm.at[pl.ds(base0, _GRP)], sem_w1).wait()
            pltpu.async_copy(
                r0, out_hbm.at[pl.ds(base0 + (_NIT2 - 1) * _GRP, _GRP)],
                sem_w0)
            pltpu.make_async_copy(
                r0, out_hbm.at[pl.ds(base0, _GRP)], sem_w0).wait()

        @pl.when(cid == 0)
        def _():
            run(glat_hbm, src_hbm, srcf_hbm)

        @pl.when(cid == 1)
        def _():
            run(mlat_hbm, dst_hbm, dstf_hbm)

    return k(glat, mlat, src, dst)


def _sc_segsum(eupd, dst, zeros_slice):
    mesh = _sc_mesh()

    @functools.partial(
        pl.kernel,
        out_type=jax.ShapeDtypeStruct((_NC, _NMP, D), jnp.float32),
        mesh=mesh,
        scratch_types=[
            pltpu.VMEM_SHARED((_NMP, D), jnp.float32),
            pltpu.VMEM((_ECH // _NW,), jnp.int32),
            pltpu.VMEM((_CS, D), jnp.float32),
            pltpu.VMEM((_CS, D), jnp.float32),
            pltpu.SemaphoreType.DMA,
        ],
    )
    def k(eupd_hbm, dst_hbm, z_hbm, agg_hbm, shared, di_v, q0, q1, sem_l):
        cid = lax.axis_index("c")
        sid = lax.axis_index("s")
        wid = sid * _NC + cid
        pltpu.sync_copy(z_hbm, shared.at[pl.ds(sid * _MPW, _MPW)])
        plsc.subcore_barrier()
        epw = _ECH // _NW
        base0 = wid * epw
        pltpu.sync_copy(dst_hbm.at[pl.ds(base0, epw)], di_v)
        ngrp = epw // _CS

        def load(g, buf):
            pltpu.async_copy(eupd_hbm.at[pl.ds(base0 + g * _CS, _CS)], buf,
                             sem_l)

        def wait_l(buf):
            pltpu.make_async_copy(eupd_hbm.at[pl.ds(base0, _CS)], buf,
                                  sem_l).wait()

        def scat(g, buf):
            pltpu.sync_copy(buf, shared.at[di_v.at[pl.ds(g * _CS, _CS)]],
                            add=True)

        load(0, q0)

        def pairbody(p, carry):
            g0 = 2 * p
            load(g0 + 1, q1)
            wait_l(q0)
            scat(g0, q0)
            load(g0 + 2, q0)
            wait_l(q1)
            scat(g0 + 1, q1)
            return carry

        lax.fori_loop(0, ngrp // 2, pairbody, 0)
        wait_l(q0)
        scat(ngrp - 1, q0)
        plsc.subcore_barrier()
        pltpu.sync_copy(shared.at[pl.ds(sid * _MPW, _MPW)],
                        agg_hbm.at[cid, pl.ds(sid * _MPW, _MPW)])

    return k(eupd, dst, zeros_slice)


def kernel(grid_nodes_features, params, edge_index):
    b = grid_nodes_features.shape[0]
    gx = grid_nodes_features.reshape(NG, IN_GRID)

    ge = params["grid_embed"]
    me = params["mesh_embed"]
    ee = params["edge_embed"]
    em = params["edge_mlp"]
    nm = params["node_mlp"]
    gn = params["grid_node_mlp"]

    def row(v):
        return v.reshape(1, D)

    gw1 = ge["W1"]
    grid_lat = _run_rows(
        _grid_embed_body,
        [gx.T, params["grid_geo"].T], [],
        [gw1[:IN_GRID], gw1[IN_GRID:], row(ge["b1"]), ge["W2"], row(ge["b2"]),
         row(ge["ln_g"]), row(ge["ln_b"])],
        NG, 4096)

    mesh_lat = _run_rows(
        _small_embed_body,
        [params["mesh_geo"].T], [],
        [me["W1"], row(me["b1"]), me["W2"], row(me["b2"]), row(me["ln_g"]),
         row(me["ln_b"])],
        NM, 2048)

    emw1 = em["W1"]
    euw = [ee["W1"], row(ee["b1"]), ee["W2"], row(ee["b2"]), row(ee["ln_g"]),
           row(ee["ln_b"]),
           emw1[:D], emw1[D:2 * D], emw1[2 * D:], row(em["b1"]), em["W2"],
           row(em["b2"]), row(em["ln_g"]), row(em["ln_b"])]
    eft = params["edge_feats"].T
    zeros_slice = jnp.zeros((_MPW, D), jnp.float32)
    aggs = []
    for ci in range(2):
        sl = slice(ci * _ECH, (ci + 1) * _ECH)
        src_f, dst_f = _sc_gather(grid_lat, mesh_lat,
                                  edge_index[0, sl], edge_index[1, sl])
        e_upd = _run_rows(
            _edge_update_body,
            [eft[:, sl]], [src_f, dst_f], euw,
            _ECH, 4096)
        aggs.append(_sc_segsum(e_upd, edge_index[1, sl], zeros_slice))

    nw1 = nm["W1"]
    nmw = [nw1[:D], nw1[D:], row(nm["b1"]), nm["W2"], row(nm["b2"]),
           row(nm["ln_g"]), row(nm["ln_b"])]
    mesh_out = pl.pallas_call(
        _mesh_update_body,
        grid=(NM // 1000,),
        in_specs=([_rspec(1000, D),
                   pl.BlockSpec((1, 1000, D), lambda i: (0, i, 0)),
                   pl.BlockSpec((1, 1000, D), lambda i: (1, i, 0)),
                   pl.BlockSpec((1, 1000, D), lambda i: (0, i, 0)),
                   pl.BlockSpec((1, 1000, D), lambda i: (1, i, 0))]
                  + [_wspec(*w.shape) for w in nmw]),
        out_specs=_rspec(1000, D),
        out_shape=jax.ShapeDtypeStruct((NM, D), jnp.float32),
    )(mesh_lat, aggs[0], aggs[0], aggs[1], aggs[1], *nmw)

    grid_out = _run_rows(
        _grid_out_body,
        [], [grid_lat],
        [gn["W1"], row(gn["b1"]), gn["W2"], row(gn["b2"]), row(gn["ln_g"]),
         row(gn["ln_b"])],
        NG, 4096)

    return (grid_out.reshape(b, NG, D), mesh_out.reshape(b, NM, D))


# speedup vs baseline: 1.1923x; 1.0309x over previous
import functools

import jax
import jax.numpy as jnp
from jax import lax
from jax.experimental import pallas as pl
from jax.experimental.pallas import tpu as pltpu
from jax.experimental.pallas import tpu_sc as plsc

D = 128
NG = 50000
NM = 10000
E = 320000
IN_GRID = 96
GEO = 32

_NC = 2
_NS = 16
_NW = _NC * _NS
_EPW = E // _NW
_C = 80
_G = 5
_GRP = _C * _G
_NIT = _EPW // _GRP
_NMP = 10240
_MPW = _NMP // _NS
_CS = 128
_ECHP = 163840


def _ln(y, g, b):
    mu = jnp.mean(y, axis=-1, keepdims=True)
    var = jnp.mean((y - mu) * (y - mu), axis=-1, keepdims=True)
    return (y - mu) * lax.rsqrt(var + 1e-5) * g + b


def _silu(x):
    return x * jax.nn.sigmoid(x)


def _dot(a, b):
    return jnp.dot(a, b, preferred_element_type=jnp.float32)


def _dott(at, b):
    return lax.dot_general(at, b, (((0,), (0,)), ((), ())),
                           preferred_element_type=jnp.float32)


def _grid_embed_body(xt_ref, gt_ref, w1x_ref, w1g_ref, b1_ref, w2_ref, b2_ref,
                     lg_ref, lb_ref, o_ref):
    h = _silu(_dott(xt_ref[...], w1x_ref[...])
              + _dott(gt_ref[...], w1g_ref[...]) + b1_ref[...])
    y = _dot(h, w2_ref[...]) + b2_ref[...]
    o_ref[...] = _ln(y, lg_ref[...], lb_ref[...])


def _small_embed_body(xt_ref, w1_ref, b1_ref, w2_ref, b2_ref, lg_ref, lb_ref,
                      o_ref):
    h = _silu(_dott(xt_ref[...], w1_ref[...]) + b1_ref[...])
    y = _dot(h, w2_ref[...]) + b2_ref[...]
    o_ref[...] = _ln(y, lg_ref[...], lb_ref[...])


def _edge_update_body(eft_ref, sf_ref, df_ref,
                      ew1_ref, eb1_ref, ew2_ref, eb2_ref, elg_ref, elb_ref,
                      mw1e_ref, mw1s_ref, mw1d_ref, mb1_ref, mw2_ref, mb2_ref,
                      mlg_ref, mlb_ref, o_ref):
    h0 = _silu(_dott(eft_ref[...], ew1_ref[...]) + eb1_ref[...])
    e0 = _ln(_dot(h0, ew2_ref[...]) + eb2_ref[...], elg_ref[...], elb_ref[...])
    h = _silu(_dot(e0, mw1e_ref[...]) + _dot(sf_ref[...], mw1s_ref[...])
              + _dot(df_ref[...], mw1d_ref[...]) + mb1_ref[...])
    y = _dot(h, mw2_ref[...]) + mb2_ref[...]
    o_ref[...] = e0 + _ln(y, mlg_ref[...], mlb_ref[...])


def _mesh_update_body(m_ref, a0_ref, a1_ref, a2_ref, a3_ref, w1m_ref,
                      w1a_ref, b1_ref, w2_ref, b2_ref, lg_ref, lb_ref, o_ref):
    a = (a0_ref[0] + a1_ref[0]) + (a2_ref[0] + a3_ref[0])
    h = _silu(_dot(m_ref[...], w1m_ref[...]) + _dot(a, w1a_ref[...])
              + b1_ref[...])
    y = _dot(h, w2_ref[...]) + b2_ref[...]
    o_ref[...] = m_ref[...] + _ln(y, lg_ref[...], lb_ref[...])


def _grid_out_body(g_ref, w1_ref, b1_ref, w2_ref, b2_ref, lg_ref, lb_ref,
                   o_ref):
    h = _silu(_dot(g_ref[...], w1_ref[...]) + b1_ref[...])
    y = _dot(h, w2_ref[...]) + b2_ref[...]
    o_ref[...] = _ln(y, lg_ref[...], lb_ref[...]) + g_ref[...]


def _wspec(*dims):
    return pl.BlockSpec(dims, lambda i: (0,) * len(dims))


def _rspec(r, c):
    return pl.BlockSpec((r, c), lambda i: (i, 0))


def _cspec(c, r):
    return pl.BlockSpec((c, r), lambda i: (0, i))


def _run_rows(body, col_ins, row_ins, w_ins, n_rows, r):
    in_specs = ([_cspec(a.shape[0], r) for a in col_ins]
                + [_rspec(r, a.shape[1]) for a in row_ins]
                + [_wspec(*w.shape) for w in w_ins])
    return pl.pallas_call(
        body,
        grid=(pl.cdiv(n_rows, r),),
        in_specs=in_specs,
        out_specs=_rspec(r, D),
        out_shape=jax.ShapeDtypeStruct((n_rows, D), jnp.float32),
    )(*col_ins, *row_ins, *w_ins)


def _sc_mesh():
    return plsc.VectorSubcoreMesh(core_axis_name="c", subcore_axis_name="s",
                                  num_cores=_NC, num_subcores=_NS)


_ECH = E // 2
_EPT = _ECH // _NS
_NIT2 = _EPT // _GRP


def _sc_gather(glat, mlat, src, dst):
    mesh = _sc_mesh()

    @functools.partial(
        pl.kernel,
        out_type=(jax.ShapeDtypeStruct((_ECH, D), jnp.float32),
                  jax.ShapeDtypeStruct((_ECH, D), jnp.float32)),
        mesh=mesh,
        scratch_types=[
            pltpu.VMEM((_EPT,), jnp.int32),
            pltpu.VMEM((_GRP, D), jnp.float32),
            pltpu.VMEM((_GRP, D), jnp.float32),
            pltpu.SemaphoreType.DMA,
            pltpu.SemaphoreType.DMA,
            pltpu.SemaphoreType.DMA,
        ],
    )
    def k(glat_hbm, mlat_hbm, src_hbm, dst_hbm, srcf_hbm, dstf_hbm,
          idx_v, r0, r1, sem_g, sem_w0, sem_w1):
        cid = lax.axis_index("c")
        sid = lax.axis_index("s")
        base0 = sid * _EPT

        def fire(tab, g, buf):
            for j in range(_G):
                pltpu.async_copy(
                    tab.at[idx_v.at[pl.ds(g * _GRP + j * _C, _C)]],
                    buf.at[pl.ds(j * _C, _C)], sem_g)

        def wait_g(tab, buf):
            for j in range(_G):
                pltpu.make_async_copy(
                    tab.at[idx_v.at[pl.ds(j * _C, _C)]],
                    buf.at[pl.ds(j * _C, _C)], sem_g).wait()

        def run(tab, idx_hbm, out_hbm):
            pltpu.sync_copy(idx_hbm.at[pl.ds(base0, _EPT)], idx_v)
            fire(tab, 0, r0)

            def pair(p, carry):
                g0 = 2 * p
                wait_g(tab, r0)
                pltpu.async_copy(r0, out_hbm.at[pl.ds(base0 + g0 * _GRP,
                                                      _GRP)], sem_w0)

                @pl.when(p > 0)
                def _():
                    pltpu.make_async_copy(
                        r1, out_hbm.at[pl.ds(base0, _GRP)], sem_w1).wait()

                fire(tab, g0 + 1, r1)
                wait_g(tab, r1)
                pltpu.make_async_copy(
                    r0, out_hbm.at[pl.ds(base0, _GRP)], sem_w0).wait()

                fire(tab, g0 + 2, r0)
                pltpu.async_copy(r1, out_hbm.at[pl.ds(base0 + (g0 + 1) * _GRP,
                                                      _GRP)], sem_w1)
                return carry

            lax.fori_loop(0, _NIT2 // 2, pair, 0)
            wait_g(tab, r0)
            pltpu.make_async_copy(
                r1, out_hbm.at[pl.ds(base0, _GRP)], sem_w1).wait()
            pltpu.async_copy(
                r0, out_hbm.at[pl.ds(base0 + (_NIT2 - 1) * _GRP, _GRP)],
                sem_w0)
            pltpu.make_async_copy(
                r0, out_hbm.at[pl.ds(base0, _GRP)], sem_w0).wait()

        @pl.when(cid == 0)
        def _():
            run(glat_hbm, src_hbm, srcf_hbm)

        @pl.when(cid == 1)
        def _():
            run(mlat_hbm, dst_hbm, dstf_hbm)

    return k(glat, mlat, src, dst)


def _sc_segsum(eupd, dst, zeros_slice):
    mesh = _sc_mesh()

    @functools.partial(
        pl.kernel,
        out_type=jax.ShapeDtypeStruct((_NC, _NMP, D), jnp.float32),
        mesh=mesh,
        scratch_types=[
            pltpu.VMEM_SHARED((_NMP, D), jnp.float32),
            pltpu.VMEM((_ECHP // _NW,), jnp.int32),
            pltpu.VMEM((_CS, D), jnp.float32),
            pltpu.VMEM((_CS, D), jnp.float32),
            pltpu.SemaphoreType.DMA,
        ],
    )
    def k(eupd_hbm, dst_hbm, z_hbm, agg_hbm, shared, di_v, q0, q1, sem_l):
        cid = lax.axis_index("c")
        sid = lax.axis_index("s")
        wid = sid * _NC + cid
        pltpu.sync_copy(z_hbm, shared.at[pl.ds(sid * _MPW, _MPW)])
        plsc.subcore_barrier()
        epw = _ECHP // _NW
        base0 = wid * epw
        pltpu.sync_copy(dst_hbm.at[pl.ds(base0, epw)], di_v)
        ngrp = epw // _CS

        def load(g, buf):
            pltpu.async_copy(eupd_hbm.at[pl.ds(base0 + g * _CS, _CS)], buf,
                             sem_l)

        def wait_l(buf):
            pltpu.make_async_copy(eupd_hbm.at[pl.ds(base0, _CS)], buf,
                                  sem_l).wait()

        def scat(g, buf):
            pltpu.sync_copy(buf, shared.at[di_v.at[pl.ds(g * _CS, _CS)]],
                            add=True)

        load(0, q0)

        def pairbody(p, carry):
            g0 = 2 * p
            load(g0 + 1, q1)
            wait_l(q0)
            scat(g0, q0)

            @pl.when(g0 + 2 < ngrp)
            def _():
                load(g0 + 2, q0)

            wait_l(q1)
            scat(g0 + 1, q1)
            return carry

        lax.fori_loop(0, ngrp // 2, pairbody, 0)
        if ngrp % 2:
            wait_l(q0)
            scat(ngrp - 1, q0)
        plsc.subcore_barrier()
        pltpu.sync_copy(shared.at[pl.ds(sid * _MPW, _MPW)],
                        agg_hbm.at[cid, pl.ds(sid * _MPW, _MPW)])

    return k(eupd, dst, zeros_slice)


def kernel(grid_nodes_features, params, edge_index):
    b = grid_nodes_features.shape[0]
    gx = grid_nodes_features.reshape(NG, IN_GRID)

    ge = params["grid_embed"]
    me = params["mesh_embed"]
    ee = params["edge_embed"]
    em = params["edge_mlp"]
    nm = params["node_mlp"]
    gn = params["grid_node_mlp"]

    def row(v):
        return v.reshape(1, D)

    gw1 = ge["W1"]
    grid_lat = _run_rows(
        _grid_embed_body,
        [gx.T, params["grid_geo"].T], [],
        [gw1[:IN_GRID], gw1[IN_GRID:], row(ge["b1"]), ge["W2"], row(ge["b2"]),
         row(ge["ln_g"]), row(ge["ln_b"])],
        NG, 4096)

    mesh_lat = _run_rows(
        _small_embed_body,
        [params["mesh_geo"].T], [],
        [me["W1"], row(me["b1"]), me["W2"], row(me["b2"]), row(me["ln_g"]),
         row(me["ln_b"])],
        NM, 2048)

    emw1 = em["W1"]
    euw = [ee["W1"], row(ee["b1"]), ee["W2"], row(ee["b2"]), row(ee["ln_g"]),
           row(ee["ln_b"]),
           emw1[:D], emw1[D:2 * D], emw1[2 * D:], row(em["b1"]), em["W2"],
           row(em["b2"]), row(em["ln_g"]), row(em["ln_b"])]
    eft = params["edge_feats"].T
    zeros_slice = jnp.zeros((_MPW, D), jnp.float32)
    aggs = []
    dpad = jnp.full((_ECHP - _ECH,), NM, jnp.int32)
    for ci in range(2):
        sl = slice(ci * _ECH, (ci + 1) * _ECH)
        src_f, dst_f = _sc_gather(grid_lat, mesh_lat,
                                  edge_index[0, sl], edge_index[1, sl])
        e_upd = _run_rows(
            _edge_update_body,
            [eft[:, sl]], [src_f, dst_f], euw,
            _ECHP, 4096)
        dst_p = jnp.concatenate([edge_index[1, sl], dpad])
        aggs.append(_sc_segsum(e_upd, dst_p, zeros_slice))

    nw1 = nm["W1"]
    nmw = [nw1[:D], nw1[D:], row(nm["b1"]), nm["W2"], row(nm["b2"]),
           row(nm["ln_g"]), row(nm["ln_b"])]
    mesh_out = pl.pallas_call(
        _mesh_update_body,
        grid=(NM // 1000,),
        in_specs=([_rspec(1000, D),
                   pl.BlockSpec((1, 1000, D), lambda i: (0, i, 0)),
                   pl.BlockSpec((1, 1000, D), lambda i: (1, i, 0)),
                   pl.BlockSpec((1, 1000, D), lambda i: (0, i, 0)),
                   pl.BlockSpec((1, 1000, D), lambda i: (1, i, 0))]
                  + [_wspec(*w.shape) for w in nmw]),
        out_specs=_rspec(1000, D),
        out_shape=jax.ShapeDtypeStruct((NM, D), jnp.float32),
    )(mesh_lat, aggs[0], aggs[0], aggs[1], aggs[1], *nmw)

    grid_out = _run_rows(
        _grid_out_body,
        [], [grid_lat],
        [gn["W1"], row(gn["b1"]), gn["W2"], row(gn["b2"]), row(gn["ln_g"]),
         row(gn["ln_b"])],
        NG, 4096)

    return (grid_out.reshape(b, NG, D), mesh_out.reshape(b, NM, D))


# speedup vs baseline: 1.1930x; 1.0006x over previous
import functools

import jax
import jax.numpy as jnp
from jax import lax
from jax.experimental import pallas as pl
from jax.experimental.pallas import tpu as pltpu
from jax.experimental.pallas import tpu_sc as plsc

D = 128
NG = 50000
NM = 10000
E = 320000
IN_GRID = 96
GEO = 32

_NC = 2
_NS = 16
_NW = _NC * _NS
_EPW = E // _NW
_C = 40
_G = 10
_GRP = _C * _G
_NIT = _EPW // _GRP
_NMP = 10240
_MPW = _NMP // _NS
_CS = 128
_ECHP = 163840


def _ln(y, g, b):
    mu = jnp.mean(y, axis=-1, keepdims=True)
    var = jnp.mean((y - mu) * (y - mu), axis=-1, keepdims=True)
    return (y - mu) * lax.rsqrt(var + 1e-5) * g + b


def _silu(x):
    return x * jax.nn.sigmoid(x)


def _dot(a, b):
    return jnp.dot(a, b, preferred_element_type=jnp.float32)


def _dott(at, b):
    return lax.dot_general(at, b, (((0,), (0,)), ((), ())),
                           preferred_element_type=jnp.float32)


def _grid_embed_body(xt_ref, gt_ref, w1x_ref, w1g_ref, b1_ref, w2_ref, b2_ref,
                     lg_ref, lb_ref, o_ref):
    h = _silu(_dott(xt_ref[...], w1x_ref[...])
              + _dott(gt_ref[...], w1g_ref[...]) + b1_ref[...])
    y = _dot(h, w2_ref[...]) + b2_ref[...]
    o_ref[...] = _ln(y, lg_ref[...], lb_ref[...])


def _small_embed_body(xt_ref, w1_ref, b1_ref, w2_ref, b2_ref, lg_ref, lb_ref,
                      o_ref):
    h = _silu(_dott(xt_ref[...], w1_ref[...]) + b1_ref[...])
    y = _dot(h, w2_ref[...]) + b2_ref[...]
    o_ref[...] = _ln(y, lg_ref[...], lb_ref[...])


def _edge_update_body(eft_ref, sf_ref, df_ref,
                      ew1_ref, eb1_ref, ew2_ref, eb2_ref, elg_ref, elb_ref,
                      mw1e_ref, mw1s_ref, mw1d_ref, mb1_ref, mw2_ref, mb2_ref,
                      mlg_ref, mlb_ref, o_ref):
    h0 = _silu(_dott(eft_ref[...], ew1_ref[...]) + eb1_ref[...])
    e0 = _ln(_dot(h0, ew2_ref[...]) + eb2_ref[...], elg_ref[...], elb_ref[...])
    h = _silu(_dot(e0, mw1e_ref[...]) + _dot(sf_ref[...], mw1s_ref[...])
              + _dot(df_ref[...], mw1d_ref[...]) + mb1_ref[...])
    y = _dot(h, mw2_ref[...]) + mb2_ref[...]
    o_ref[...] = e0 + _ln(y, mlg_ref[...], mlb_ref[...])


def _mesh_update_body(m_ref, a0_ref, a1_ref, a2_ref, a3_ref, w1m_ref,
                      w1a_ref, b1_ref, w2_ref, b2_ref, lg_ref, lb_ref, o_ref):
    a = (a0_ref[0] + a1_ref[0]) + (a2_ref[0] + a3_ref[0])
    h = _silu(_dot(m_ref[...], w1m_ref[...]) + _dot(a, w1a_ref[...])
              + b1_ref[...])
    y = _dot(h, w2_ref[...]) + b2_ref[...]
    o_ref[...] = m_ref[...] + _ln(y, lg_ref[...], lb_ref[...])


def _grid_out_body(g_ref, w1_ref, b1_ref, w2_ref, b2_ref, lg_ref, lb_ref,
                   o_ref):
    h = _silu(_dot(g_ref[...], w1_ref[...]) + b1_ref[...])
    y = _dot(h, w2_ref[...]) + b2_ref[...]
    o_ref[...] = _ln(y, lg_ref[...], lb_ref[...]) + g_ref[...]


def _wspec(*dims):
    return pl.BlockSpec(dims, lambda i: (0,) * len(dims))


def _rspec(r, c):
    return pl.BlockSpec((r, c), lambda i: (i, 0))


def _cspec(c, r):
    return pl.BlockSpec((c, r), lambda i: (0, i))


def _run_rows(body, col_ins, row_ins, w_ins, n_rows, r):
    in_specs = ([_cspec(a.shape[0], r) for a in col_ins]
                + [_rspec(r, a.shape[1]) for a in row_ins]
                + [_wspec(*w.shape) for w in w_ins])
    return pl.pallas_call(
        body,
        grid=(pl.cdiv(n_rows, r),),
        in_specs=in_specs,
        out_specs=_rspec(r, D),
        out_shape=jax.ShapeDtypeStruct((n_rows, D), jnp.float32),
    )(*col_ins, *row_ins, *w_ins)


def _sc_mesh():
    return plsc.VectorSubcoreMesh(core_axis_name="c", subcore_axis_name="s",
                                  num_cores=_NC, num_subcores=_NS)


_ECH = E // 2
_EPT = _ECH // _NS
_NIT2 = _EPT // _GRP


def _sc_gather(glat, mlat, src, dst):
    mesh = _sc_mesh()

    @functools.partial(
        pl.kernel,
        out_type=(jax.ShapeDtypeStruct((_ECH, D), jnp.float32),
                  jax.ShapeDtypeStruct((_ECH, D), jnp.float32)),
        mesh=mesh,
        scratch_types=[
            pltpu.VMEM((_EPT,), jnp.int32),
            pltpu.VMEM((_GRP, D), jnp.float32),
            pltpu.VMEM((_GRP, D), jnp.float32),
            pltpu.SemaphoreType.DMA,
            pltpu.SemaphoreType.DMA,
            pltpu.SemaphoreType.DMA,
        ],
    )
    def k(glat_hbm, mlat_hbm, src_hbm, dst_hbm, srcf_hbm, dstf_hbm,
          idx_v, r0, r1, sem_g, sem_w0, sem_w1):
        cid = lax.axis_index("c")
        sid = lax.axis_index("s")
        base0 = sid * _EPT

        def fire(tab, g, buf):
            for j in range(_G):
                pltpu.async_copy(
                    tab.at[idx_v.at[pl.ds(g * _GRP + j * _C, _C)]],
                    buf.at[pl.ds(j * _C, _C)], sem_g)

        def wait_g(tab, buf):
            for j in range(_G):
                pltpu.make_async_copy(
                    tab.at[idx_v.at[pl.ds(j * _C, _C)]],
                    buf.at[pl.ds(j * _C, _C)], sem_g).wait()

        def run(tab, idx_hbm, out_hbm):
            pltpu.sync_copy(idx_hbm.at[pl.ds(base0, _EPT)], idx_v)
            fire(tab, 0, r0)

            def pair(p, carry):
                g0 = 2 * p
                wait_g(tab, r0)
                pltpu.async_copy(r0, out_hbm.at[pl.ds(base0 + g0 * _GRP,
                                                      _GRP)], sem_w0)

                @pl.when(p > 0)
                def _():
                    pltpu.make_async_copy(
                        r1, out_hbm.at[pl.ds(base0, _GRP)], sem_w1).wait()

                fire(tab, g0 + 1, r1)
                wait_g(tab, r1)
                pltpu.make_async_copy(
                    r0, out_hbm.at[pl.ds(base0, _GRP)], sem_w0).wait()

                fire(tab, g0 + 2, r0)
                pltpu.async_copy(r1, out_hbm.at[pl.ds(base0 + (g0 + 1) * _GRP,
                                                      _GRP)], sem_w1)
                return carry

            lax.fori_loop(0, _NIT2 // 2, pair, 0)
            wait_g(tab, r0)
            pltpu.make_async_copy(
                r1, out_hbm.at[pl.ds(base0, _GRP)], sem_w1).wait()
            pltpu.async_copy(
                r0, out_hbm.at[pl.ds(base0 + (_NIT2 - 1) * _GRP, _GRP)],
                sem_w0)
            pltpu.make_async_copy(
                r0, out_hbm.at[pl.ds(base0, _GRP)], sem_w0).wait()

        @pl.when(cid == 0)
        def _():
            run(glat_hbm, src_hbm, srcf_hbm)

        @pl.when(cid == 1)
        def _():
            run(mlat_hbm, dst_hbm, dstf_hbm)

    return k(glat, mlat, src, dst)


def _sc_segsum(eupd, dst, zeros_slice):
    mesh = _sc_mesh()

    @functools.partial(
        pl.kernel,
        out_type=jax.ShapeDtypeStruct((_NC, _NMP, D), jnp.float32),
        mesh=mesh,
        scratch_types=[
            pltpu.VMEM_SHARED((_NMP, D), jnp.float32),
            pltpu.VMEM((_ECHP // _NW,), jnp.int32),
            pltpu.VMEM((_CS, D), jnp.float32),
            pltpu.VMEM((_CS, D), jnp.float32),
            pltpu.SemaphoreType.DMA,
        ],
    )
    def k(eupd_hbm, dst_hbm, z_hbm, agg_hbm, shared, di_v, q0, q1, sem_l):
        cid = lax.axis_index("c")
        sid = lax.axis_index("s")
        wid = sid * _NC + cid
        pltpu.sync_copy(z_hbm, shared.at[pl.ds(sid * _MPW, _MPW)])
        plsc.subcore_barrier()
        epw = _ECHP // _NW
        base0 = wid * epw
        pltpu.sync_copy(dst_hbm.at[pl.ds(base0, epw)], di_v)
        ngrp = epw // _CS

        def load(g, buf):
            pltpu.async_copy(eupd_hbm.at[pl.ds(base0 + g * _CS, _CS)], buf,
                             sem_l)

        def wait_l(buf):
            pltpu.make_async_copy(eupd_hbm.at[pl.ds(base0, _CS)], buf,
                                  sem_l).wait()

        def scat(g, buf):
            pltpu.sync_copy(buf, shared.at[di_v.at[pl.ds(g * _CS, _CS)]],
                            add=True)

        load(0, q0)

        def pairbody(p, carry):
            g0 = 2 * p
            load(g0 + 1, q1)
            wait_l(q0)
            scat(g0, q0)

            @pl.when(g0 + 2 < ngrp)
            def _():
                load(g0 + 2, q0)

            wait_l(q1)
            scat(g0 + 1, q1)
            return carry

        lax.fori_loop(0, ngrp // 2, pairbody, 0)
        if ngrp % 2:
            wait_l(q0)
            scat(ngrp - 1, q0)
        plsc.subcore_barrier()
        pltpu.sync_copy(shared.at[pl.ds(sid * _MPW, _MPW)],
                        agg_hbm.at[cid, pl.ds(sid * _MPW, _MPW)])

    return k(eupd, dst, zeros_slice)


def kernel(grid_nodes_features, params, edge_index):
    b = grid_nodes_features.shape[0]
    gx = grid_nodes_features.reshape(NG, IN_GRID)

    ge = params["grid_embed"]
    me = params["mesh_embed"]
    ee = params["edge_embed"]
    em = params["edge_mlp"]
    nm = params["node_mlp"]
    gn = params["grid_node_mlp"]

    def row(v):
        return v.reshape(1, D)

    gw1 = ge["W1"]
    grid_lat = _run_rows(
        _grid_embed_body,
        [gx.T, params["grid_geo"].T], [],
        [gw1[:IN_GRID], gw1[IN_GRID:], row(ge["b1"]), ge["W2"], row(ge["b2"]),
         row(ge["ln_g"]), row(ge["ln_b"])],
        NG, 4096)

    mesh_lat = _run_rows(
        _small_embed_body,
        [params["mesh_geo"].T], [],
        [me["W1"], row(me["b1"]), me["W2"], row(me["b2"]), row(me["ln_g"]),
         row(me["ln_b"])],
        NM, 2048)

    emw1 = em["W1"]
    euw = [ee["W1"], row(ee["b1"]), ee["W2"], row(ee["b2"]), row(ee["ln_g"]),
           row(ee["ln_b"]),
           emw1[:D], emw1[D:2 * D], emw1[2 * D:], row(em["b1"]), em["W2"],
           row(em["b2"]), row(em["ln_g"]), row(em["ln_b"])]
    eft = params["edge_feats"].T
    zeros_slice = jnp.zeros((_MPW, D), jnp.float32)
    aggs = []
    dpad = jnp.full((_ECHP - _ECH,), NM, jnp.int32)
    for ci in range(2):
        sl = slice(ci * _ECH, (ci + 1) * _ECH)
        src_f, dst_f = _sc_gather(grid_lat, mesh_lat,
                                  edge_index[0, sl], edge_index[1, sl])
        e_upd = _run_rows(
            _edge_update_body,
            [eft[:, sl]], [src_f, dst_f], euw,
            _ECHP, 4096)
        dst_p = jnp.concatenate([edge_index[1, sl], dpad])
        aggs.append(_sc_segsum(e_upd, dst_p, zeros_slice))

    nw1 = nm["W1"]
    nmw = [nw1[:D], nw1[D:], row(nm["b1"]), nm["W2"], row(nm["b2"]),
           row(nm["ln_g"]), row(nm["ln_b"])]
    mesh_out = pl.pallas_call(
        _mesh_update_body,
        grid=(NM // 1000,),
        in_specs=([_rspec(1000, D),
                   pl.BlockSpec((1, 1000, D), lambda i: (0, i, 0)),
                   pl.BlockSpec((1, 1000, D), lambda i: (1, i, 0)),
                   pl.BlockSpec((1, 1000, D), lambda i: (0, i, 0)),
                   pl.BlockSpec((1, 1000, D), lambda i: (1, i, 0))]
                  + [_wspec(*w.shape) for w in nmw]),
        out_specs=_rspec(1000, D),
        out_shape=jax.ShapeDtypeStruct((NM, D), jnp.float32),
    )(mesh_lat, aggs[0], aggs[0], aggs[1], aggs[1], *nmw)

    grid_out = _run_rows(
        _grid_out_body,
        [], [grid_lat],
        [gn["W1"], row(gn["b1"]), gn["W2"], row(gn["b2"]), row(gn["ln_g"]),
         row(gn["ln_b"])],
        NG, 4096)

    return (grid_out.reshape(b, NG, D), mesh_out.reshape(b, NM, D))
